# baseline (speedup 1.0000x reference)
import functools

import jax
import jax.numpy as jnp
from jax import lax
from jax.experimental import pallas as pl
from jax.experimental.pallas import tpu as pltpu
from jax.experimental.pallas import tpu_sc as plsc

B = 16384
D = 32
F = 32
NC, NS, L = 2, 16, 16
NW = NC * NS
BPW = B // NW
CHUNK = 128
NCHUNK = BPW // CHUNK
TBLK = 4096


def _transpose_table(tt):
    n = tt.shape[1]
    grid = (n + TBLK - 1) // TBLK

    def body(x_ref, o_ref):
        eye = jnp.eye(D, dtype=jnp.float32)
        o_ref[...] = jax.lax.dot_general(
            x_ref[...], eye, (((0,), (0,)), ((), ())),
            precision=jax.lax.Precision.HIGHEST)

    return pl.pallas_call(
        body,
        grid=(grid,),
        in_specs=[pl.BlockSpec((D, TBLK), lambda i: (0, i))],
        out_specs=pl.BlockSpec((TBLK, D), lambda i: (i, 0)),
        out_shape=jax.ShapeDtypeStruct((n, D), jnp.float32),
    )(tt)


def _make_sc_kernel():
    mesh = plsc.VectorSubcoreMesh(core_axis_name="c", subcore_axis_name="s")
    cp = pltpu.CompilerParams(needs_layout_passes=False,
                              use_tc_tiling_on_sc=False)

    @functools.partial(
        pl.kernel,
        mesh=mesh,
        out_type=jax.ShapeDtypeStruct((B,), jnp.float32),
        scratch_types=[
            pltpu.VMEM((NCHUNK, CHUNK), jnp.int32),
            pltpu.VMEM((NCHUNK, CHUNK), jnp.int32),
            pltpu.VMEM((BPW, D), jnp.float32),
            pltpu.VMEM((BPW, D), jnp.float32),
            pltpu.VMEM((BPW, F), jnp.float32),
            pltpu.VMEM((80,), jnp.float32),
            pltpu.VMEM((BPW,), jnp.float32),
            pltpu.SemaphoreType.DMA,
        ],
        compiler_params=cp,
    )
    def k(ui_hbm, mi_hbm, mf_hbm, ut_hbm, mt_hbm, wb_hbm, o_hbm,
          ui_v, mi_v, ue_v, me_v, mf_v, wb_v, o_v, sem):
        wid = lax.axis_index("s") * NC + lax.axis_index("c")
        base = wid * BPW

        pltpu.sync_copy(ui_hbm.at[pl.ds(wid * NCHUNK, NCHUNK)], ui_v)
        pltpu.sync_copy(mi_hbm.at[pl.ds(wid * NCHUNK, NCHUNK)], mi_v)

        copies = []
        for j in range(NCHUNK):
            copies.append(pltpu.async_copy(
                ut_hbm.at[ui_v.at[j]], ue_v.at[pl.ds(j * CHUNK, CHUNK)], sem))
            copies.append(pltpu.async_copy(
                mt_hbm.at[mi_v.at[j]], me_v.at[pl.ds(j * CHUNK, CHUNK)], sem))
        pltpu.sync_copy(mf_hbm.at[pl.ds(base, BPW)], mf_v)
        pltpu.sync_copy(wb_hbm, wb_v)
        for c in copies:
            c.wait()

        w1a = wb_v[pl.ds(0, L)]
        w1b = wb_v[pl.ds(L, L)]
        w2a = wb_v[pl.ds(2 * L, L)]
        w2b = wb_v[pl.ds(3 * L, L)]
        bias = wb_v[pl.ds(4 * L, L)][0]
        lanes = lax.iota(jnp.int32, L)

        @pl.loop(0, BPW // L)
        def _(g):
            r0 = g * L
            res = jnp.zeros((L,), jnp.float32)
            for kk in range(L):
                i = r0 + kk
                v = (ue_v[i, pl.ds(0, L)] * me_v[i, pl.ds(0, L)] * w1a
                     + ue_v[i, pl.ds(L, L)] * me_v[i, pl.ds(L, L)] * w1b
                     + mf_v[i, pl.ds(0, L)] * w2a
                     + mf_v[i, pl.ds(L, L)] * w2b)
                res = jnp.where(lanes == kk, jnp.sum(v), res)
            o_v[pl.ds(r0, L)] = res + bias

        pltpu.sync_copy(o_v, o_hbm.at[pl.ds(base, BPW)])

    return k


_sc_forward = _make_sc_kernel()


def kernel(user_idx, movie_idx, movie_feats, user_table, movie_table, W, b):
    ui = user_idx.astype(jnp.int32).reshape(B // CHUNK, CHUNK)
    mi = movie_idx.astype(jnp.int32).reshape(B // CHUNK, CHUNK)
    wb = jnp.zeros((80,), jnp.float32).at[:64].set(W[:, 0]).at[64].set(b[0])
    ut = _transpose_table(user_table.T)
    mt = _transpose_table(movie_table.T)
    return _sc_forward(ui, mi, movie_feats, ut, mt, wb)


# speedup vs baseline: 1.2247x; 1.2247x over previous
import functools

import jax
import jax.numpy as jnp
from jax import lax
from jax.experimental import pallas as pl
from jax.experimental.pallas import tpu as pltpu
from jax.experimental.pallas import tpu_sc as plsc

B = 16384
D = 32
F = 32
NC, NS, L = 2, 16, 16
NW = NC * NS
BPW = B // NW
CHUNK = 128
NCHUNK = BPW // CHUNK
TBLK = 4096


def _transpose_table(tt):
    n = tt.shape[1]
    grid = (n + TBLK - 1) // TBLK

    def body(x_ref, o_ref):
        eye = jnp.eye(D, dtype=jnp.float32)
        o_ref[...] = jax.lax.dot_general(
            x_ref[...], eye, (((0,), (0,)), ((), ())))

    return pl.pallas_call(
        body,
        grid=(grid,),
        in_specs=[pl.BlockSpec((D, TBLK), lambda i: (0, i))],
        out_specs=pl.BlockSpec((TBLK, D), lambda i: (i, 0)),
        out_shape=jax.ShapeDtypeStruct((n, D), jnp.float32),
        compiler_params=pltpu.CompilerParams(
            fuse_transposed_lhs_in_matmul=True),
    )(tt)


def _make_sc_kernel():
    mesh = plsc.VectorSubcoreMesh(core_axis_name="c", subcore_axis_name="s")
    cp = pltpu.CompilerParams(needs_layout_passes=False,
                              use_tc_tiling_on_sc=False)

    @functools.partial(
        pl.kernel,
        mesh=mesh,
        out_type=jax.ShapeDtypeStruct((B,), jnp.float32),
        scratch_types=[
            pltpu.VMEM((NCHUNK, CHUNK), jnp.int32),
            pltpu.VMEM((NCHUNK, CHUNK), jnp.int32),
            pltpu.VMEM((BPW, D), jnp.float32),
            pltpu.VMEM((BPW, D), jnp.float32),
            pltpu.VMEM((BPW, F), jnp.float32),
            pltpu.VMEM((80,), jnp.float32),
            pltpu.VMEM((BPW,), jnp.float32),
            pltpu.SemaphoreType.DMA,
        ],
        compiler_params=cp,
    )
    def k(ui_hbm, mi_hbm, mf_hbm, ut_hbm, mt_hbm, wb_hbm, o_hbm,
          ui_v, mi_v, ue_v, me_v, mf_v, wb_v, o_v, sem):
        wid = lax.axis_index("s") * NC + lax.axis_index("c")
        base = wid * BPW

        pltpu.sync_copy(ui_hbm.at[pl.ds(wid * NCHUNK, NCHUNK)], ui_v)
        pltpu.sync_copy(mi_hbm.at[pl.ds(wid * NCHUNK, NCHUNK)], mi_v)

        copies = []
        for j in range(NCHUNK):
            copies.append(pltpu.async_copy(
                ut_hbm.at[ui_v.at[j]], ue_v.at[pl.ds(j * CHUNK, CHUNK)], sem))
            copies.append(pltpu.async_copy(
                mt_hbm.at[mi_v.at[j]], me_v.at[pl.ds(j * CHUNK, CHUNK)], sem))
        pltpu.sync_copy(mf_hbm.at[pl.ds(base, BPW)], mf_v)
        pltpu.sync_copy(wb_hbm, wb_v)
        for c in copies:
            c.wait()

        w1a = wb_v[pl.ds(0, L)]
        w1b = wb_v[pl.ds(L, L)]
        w2a = wb_v[pl.ds(2 * L, L)]
        w2b = wb_v[pl.ds(3 * L, L)]
        bias = wb_v[pl.ds(4 * L, L)][0]
        lanes = lax.iota(jnp.int32, L)

        @pl.loop(0, BPW // L)
        def _(g):
            r0 = g * L
            res = jnp.zeros((L,), jnp.float32)
            for kk in range(L):
                i = r0 + kk
                v = (ue_v[i, pl.ds(0, L)] * me_v[i, pl.ds(0, L)] * w1a
                     + ue_v[i, pl.ds(L, L)] * me_v[i, pl.ds(L, L)] * w1b
                     + mf_v[i, pl.ds(0, L)] * w2a
                     + mf_v[i, pl.ds(L, L)] * w2b)
                res = jnp.where(lanes == kk, jnp.sum(v), res)
            o_v[pl.ds(r0, L)] = res + bias

        pltpu.sync_copy(o_v, o_hbm.at[pl.ds(base, BPW)])

    return k


_sc_forward = _make_sc_kernel()


def kernel(user_idx, movie_idx, movie_feats, user_table, movie_table, W, b):
    ui = user_idx.astype(jnp.int32).reshape(B // CHUNK, CHUNK)
    mi = movie_idx.astype(jnp.int32).reshape(B // CHUNK, CHUNK)
    wb = jnp.zeros((80,), jnp.float32).at[:64].set(W[:, 0]).at[64].set(b[0])
    ut = _transpose_table(user_table.T)
    mt = _transpose_table(movie_table.T)
    return _sc_forward(ui, mi, movie_feats, ut, mt, wb)


# speedup vs baseline: 1.4549x; 1.1880x over previous
import functools

import jax
import jax.numpy as jnp
from jax import lax
from jax.experimental import pallas as pl
from jax.experimental.pallas import tpu as pltpu
from jax.experimental.pallas import tpu_sc as plsc

B = 16384
D = 32
F = 32
NC, NS, L = 2, 16, 16
NW = NC * NS
BPW = B // NW
CHUNK = 128
NCHUNK = BPW // CHUNK
TBLK = 16384


def _transpose_table(tt):
    n = tt.shape[1]
    grid = (n + TBLK - 1) // TBLK

    def body(x_ref, o_ref):
        eye = jnp.eye(D, dtype=jnp.float32)
        o_ref[...] = jax.lax.dot_general(
            x_ref[...], eye, (((0,), (0,)), ((), ())))

    return pl.pallas_call(
        body,
        grid=(grid,),
        in_specs=[pl.BlockSpec((D, TBLK), lambda i: (0, i))],
        out_specs=pl.BlockSpec((TBLK, D), lambda i: (i, 0)),
        out_shape=jax.ShapeDtypeStruct((n, D), jnp.float32),
        compiler_params=pltpu.CompilerParams(
            fuse_transposed_lhs_in_matmul=True),
    )(tt)


def _make_sc_kernel():
    mesh = plsc.VectorSubcoreMesh(core_axis_name="c", subcore_axis_name="s")
    cp = pltpu.CompilerParams(needs_layout_passes=False,
                              use_tc_tiling_on_sc=False)

    @functools.partial(
        pl.kernel,
        mesh=mesh,
        out_type=jax.ShapeDtypeStruct((B,), jnp.float32),
        scratch_types=[
            pltpu.VMEM((NCHUNK, CHUNK), jnp.int32),
            pltpu.VMEM((NCHUNK, CHUNK), jnp.int32),
            pltpu.VMEM((BPW, D), jnp.float32),
            pltpu.VMEM((BPW, D), jnp.float32),
            pltpu.VMEM((BPW, F), jnp.float32),
            pltpu.VMEM((80,), jnp.float32),
            pltpu.VMEM((BPW,), jnp.float32),
            pltpu.SemaphoreType.DMA,
        ],
        compiler_params=cp,
    )
    def k(ui_hbm, mi_hbm, mf_hbm, ut_hbm, mt_hbm, wb_hbm, o_hbm,
          ui_v, mi_v, ue_v, me_v, mf_v, wb_v, o_v, sem):
        wid = lax.axis_index("s") * NC + lax.axis_index("c")
        base = wid * BPW

        pltpu.sync_copy(ui_hbm.at[pl.ds(wid * NCHUNK, NCHUNK)], ui_v)
        pltpu.sync_copy(mi_hbm.at[pl.ds(wid * NCHUNK, NCHUNK)], mi_v)

        copies = []
        for j in range(NCHUNK):
            copies.append(pltpu.async_copy(
                ut_hbm.at[ui_v.at[j]], ue_v.at[pl.ds(j * CHUNK, CHUNK)], sem))
            copies.append(pltpu.async_copy(
                mt_hbm.at[mi_v.at[j]], me_v.at[pl.ds(j * CHUNK, CHUNK)], sem))
        pltpu.sync_copy(mf_hbm.at[pl.ds(base, BPW)], mf_v)
        pltpu.sync_copy(wb_hbm, wb_v)
        for c in copies:
            c.wait()

        w1a = wb_v[pl.ds(0, L)]
        w1b = wb_v[pl.ds(L, L)]
        w2a = wb_v[pl.ds(2 * L, L)]
        w2b = wb_v[pl.ds(3 * L, L)]
        bias = wb_v[pl.ds(4 * L, L)][0]
        lanes = lax.iota(jnp.int32, L)

        @pl.loop(0, BPW // L)
        def _(g):
            r0 = g * L
            res = jnp.zeros((L,), jnp.float32)
            for kk in range(L):
                i = r0 + kk
                v = (ue_v[i, pl.ds(0, L)] * me_v[i, pl.ds(0, L)] * w1a
                     + ue_v[i, pl.ds(L, L)] * me_v[i, pl.ds(L, L)] * w1b
                     + mf_v[i, pl.ds(0, L)] * w2a
                     + mf_v[i, pl.ds(L, L)] * w2b)
                res = jnp.where(lanes == kk, jnp.sum(v), res)
            o_v[pl.ds(r0, L)] = res + bias

        pltpu.sync_copy(o_v, o_hbm.at[pl.ds(base, BPW)])

    return k


_sc_forward = _make_sc_kernel()


def kernel(user_idx, movie_idx, movie_feats, user_table, movie_table, W, b):
    ui = user_idx.astype(jnp.int32).reshape(B // CHUNK, CHUNK)
    mi = movie_idx.astype(jnp.int32).reshape(B // CHUNK, CHUNK)
    wb = jnp.zeros((80,), jnp.float32).at[:64].set(W[:, 0]).at[64].set(b[0])
    ut = _transpose_table(user_table.T)
    mt = _transpose_table(movie_table.T)
    return _sc_forward(ui, mi, movie_feats, ut, mt, wb)


# speedup vs baseline: 3.4141x; 2.3467x over previous
import functools

import jax
import jax.numpy as jnp
from jax import lax
from jax.experimental import pallas as pl
from jax.experimental.pallas import tpu as pltpu
from jax.experimental.pallas import tpu_sc as plsc

B = 16384
D = 32
F = 32
NC, NS, L = 2, 16, 16
NW = NC * NS
BPW = B // NW
CHUNK = 128
NCHUNK = BPW // CHUNK

NU = 1000001
NM = 100001
SLAB = 2048

FU = (NU // CHUNK) * CHUNK
RU = NU - FU
SU = FU + CHUNK
USLAB = FU // SLAB
UREM = FU - USLAB * SLAB

FM = (NM // CHUNK) * CHUNK
RM = NM - FM
SM = FM + CHUNK
MSLAB = FM // SLAB
MREM = FM - MSLAB * SLAB

_mesh = plsc.VectorSubcoreMesh(core_axis_name="c", subcore_axis_name="s")


def _make_detile_kernel():
    cp = pltpu.CompilerParams(needs_layout_passes=False,
                              use_tc_tiling_on_sc=True)

    @functools.partial(
        pl.kernel,
        mesh=_mesh,
        out_type=[jax.ShapeDtypeStruct((D * SU,), jnp.float32),
                  jax.ShapeDtypeStruct((D * SM,), jnp.float32)],
        scratch_types=[
            pltpu.VMEM((8, SLAB), jnp.float32),
            pltpu.VMEM((D * CHUNK,), jnp.float32),
            pltpu.SemaphoreType.DMA,
            pltpu.SemaphoreType.DMA,
        ],
        compiler_params=cp,
    )
    def k(utt_hbm, mtt_hbm, utail_hbm, mtail_hbm, uo_hbm, mo_hbm,
          buf, tbuf, rsem, wsem):
        wid = lax.axis_index("s") * NC + lax.axis_index("c")

        def move_slab(src_hbm, dst_hbm, stride, g, c0, width):
            pltpu.async_copy(
                src_hbm.at[pl.ds(8 * g, 8), pl.ds(c0, width)],
                buf.at[:, pl.ds(0, width)], rsem).wait()
            for l in range(8):
                @pl.loop(0, width // CHUNK)
                def _(c):
                    off = pl.multiple_of(c * CHUNK, CHUNK)
                    pltpu.async_copy(
                        buf.at[l, pl.ds(off, CHUNK)],
                        dst_hbm.at[pl.ds(
                            (8 * g + l) * stride + c0 + off, CHUNK)],
                        wsem)
            for l in range(8):
                @pl.loop(0, width // CHUNK)
                def _(c):
                    off = pl.multiple_of(c * CHUNK, CHUNK)
                    pltpu.make_async_copy(
                        buf.at[l, pl.ds(off, CHUNK)],
                        dst_hbm.at[pl.ds(
                            (8 * g + l) * stride + c0 + off, CHUNK)],
                        wsem).wait()

        for g in range(4):
            @pl.loop(0, (USLAB + NW - 1) // NW)
            def _(kk):
                s = wid + NW * kk

                @pl.when(s < USLAB)
                def _():
                    c0 = pl.multiple_of(s * SLAB, SLAB)
                    move_slab(utt_hbm, uo_hbm, SU, g, c0, SLAB)

        for g in range(4):
            @pl.loop(0, (MSLAB + NW - 1) // NW)
            def _(kk):
                s = wid + NW * kk

                @pl.when(s < MSLAB)
                def _():
                    c0 = pl.multiple_of(s * SLAB, SLAB)
                    move_slab(mtt_hbm, mo_hbm, SM, g, c0, SLAB)

        for g in range(4):
            @pl.when(wid == 16 + g)
            def _():
                move_slab(utt_hbm, uo_hbm, SU, g, USLAB * SLAB, UREM)

            @pl.when(wid == 20 + g)
            def _():
                move_slab(mtt_hbm, mo_hbm, SM, g, MSLAB * SLAB, MREM)

        @pl.when(wid == 24)
        def _():
            pltpu.async_copy(utail_hbm, tbuf, rsem).wait()
            for j in range(D):
                pltpu.async_copy(
                    tbuf.at[pl.ds(j * CHUNK, CHUNK)],
                    uo_hbm.at[pl.ds(j * SU + FU, CHUNK)], wsem)
            for j in range(D):
                pltpu.make_async_copy(
                    tbuf.at[pl.ds(j * CHUNK, CHUNK)],
                    uo_hbm.at[pl.ds(j * SU + FU, CHUNK)], wsem).wait()

        @pl.when(wid == 25)
        def _():
            pltpu.async_copy(mtail_hbm, tbuf, rsem).wait()
            for j in range(D):
                pltpu.async_copy(
                    tbuf.at[pl.ds(j * CHUNK, CHUNK)],
                    mo_hbm.at[pl.ds(j * SM + FM, CHUNK)], wsem)
            for j in range(D):
                pltpu.make_async_copy(
                    tbuf.at[pl.ds(j * CHUNK, CHUNK)],
                    mo_hbm.at[pl.ds(j * SM + FM, CHUNK)], wsem).wait()

    return k


def _make_gather_kernel():
    cp = pltpu.CompilerParams(needs_layout_passes=False,
                              use_tc_tiling_on_sc=False)

    @functools.partial(
        pl.kernel,
        mesh=_mesh,
        out_type=jax.ShapeDtypeStruct((B,), jnp.float32),
        scratch_types=[
            pltpu.VMEM((NCHUNK, CHUNK), jnp.int32),
            pltpu.VMEM((NCHUNK, CHUNK), jnp.int32),
            pltpu.VMEM((NCHUNK, CHUNK), jnp.int32),
            pltpu.VMEM((NCHUNK, CHUNK), jnp.int32),
            pltpu.VMEM((D, BPW), jnp.float32),
            pltpu.VMEM((D, BPW), jnp.float32),
            pltpu.VMEM((F, BPW), jnp.float32),
            pltpu.VMEM((80,), jnp.float32),
            pltpu.VMEM((BPW,), jnp.float32),
            pltpu.SemaphoreType.DMA,
        ],
        compiler_params=cp,
    )
    def k(ui_hbm, mi_hbm, mft_hbm, ut_hbm, mt_hbm, wb_hbm, o_hbm,
          ui_v, mi_v, uf_v, mg_v, ue_v, me_v, mf_v, wb_v, o_v, sem):
        wid = lax.axis_index("s") * NC + lax.axis_index("c")
        base = wid * BPW

        pltpu.sync_copy(ui_hbm.at[pl.ds(wid * NCHUNK, NCHUNK)], ui_v)
        pltpu.sync_copy(mi_hbm.at[pl.ds(wid * NCHUNK, NCHUNK)], mi_v)

        @pl.loop(0, D)
        def _(j):
            for c in range(NCHUNK):
                for s in range(CHUNK // L):
                    sl = pl.ds(s * L, L)
                    uf_v[c, sl] = ui_v[c, sl] + j * SU
                    mg_v[c, sl] = mi_v[c, sl] + j * SM
            for c in range(NCHUNK):
                dst = pl.ds(c * CHUNK, CHUNK)
                pltpu.async_copy(
                    ut_hbm.at[uf_v.at[c]], ue_v.at[j].at[dst], sem)
                pltpu.async_copy(
                    mt_hbm.at[mg_v.at[c]], me_v.at[j].at[dst], sem)
            for c in range(NCHUNK):
                dst = pl.ds(c * CHUNK, CHUNK)
                pltpu.make_async_copy(
                    ut_hbm.at[uf_v.at[c]], ue_v.at[j].at[dst], sem).wait()
                pltpu.make_async_copy(
                    mt_hbm.at[mg_v.at[c]], me_v.at[j].at[dst], sem).wait()

        pltpu.sync_copy(mft_hbm.at[:, pl.ds(base, BPW)], mf_v)
        pltpu.sync_copy(wb_hbm, wb_v)

        wvecs = [wb_v[pl.ds(g * L, L)] for g in range(5)]
        bias = wvecs[4][0]

        @pl.loop(0, BPW // L)
        def _(g):
            sl = pl.ds(g * L, L)
            acc = jnp.full((L,), bias, jnp.float32)
            for j in range(D):
                w1j = wvecs[j // L][j % L]
                acc = acc + ue_v[j, sl] * me_v[j, sl] * w1j
            for f in range(F):
                w2f = wvecs[2 + f // L][f % L]
                acc = acc + mf_v[f, sl] * w2f
            o_v[sl] = acc

        pltpu.sync_copy(o_v, o_hbm.at[pl.ds(base, BPW)])

    return k


_sc_detile = _make_detile_kernel()
_sc_forward = _make_gather_kernel()


def kernel(user_idx, movie_idx, movie_feats, user_table, movie_table, W, b):
    ui = user_idx.astype(jnp.int32).reshape(B // CHUNK, CHUNK)
    mi = movie_idx.astype(jnp.int32).reshape(B // CHUNK, CHUNK)
    wb = jnp.zeros((80,), jnp.float32).at[:64].set(W[:, 0]).at[64].set(b[0])
    utail = jnp.pad(user_table[FU:].T, ((0, 0), (0, CHUNK - RU))).reshape(-1)
    mtail = jnp.pad(movie_table[FM:].T, ((0, 0), (0, CHUNK - RM))).reshape(-1)
    uflat, mflat = _sc_detile(user_table.T, movie_table.T, utail, mtail)
    return _sc_forward(ui, mi, movie_feats.T, uflat, mflat, wb)


# speedup vs baseline: 3.5920x; 1.0521x over previous
import functools

import jax
import jax.numpy as jnp
from jax import lax
from jax.experimental import pallas as pl
from jax.experimental.pallas import tpu as pltpu
from jax.experimental.pallas import tpu_sc as plsc

B = 16384
D = 32
F = 32
NC, NS, L = 2, 16, 16
NW = NC * NS
BPW = B // NW
CHUNK = 128
NCHUNK = BPW // CHUNK

NU = 1000001
NM = 100001
SLAB = 2048

FU = (NU // CHUNK) * CHUNK
RU = NU - FU
SU = FU + CHUNK
USLAB = FU // SLAB
UREM = FU - USLAB * SLAB

FM = (NM // CHUNK) * CHUNK
RM = NM - FM
SM = FM + CHUNK
MSLAB = FM // SLAB
MREM = FM - MSLAB * SLAB

_mesh = plsc.VectorSubcoreMesh(core_axis_name="c", subcore_axis_name="s")


def _make_detile_kernel():
    cp = pltpu.CompilerParams(needs_layout_passes=False,
                              use_tc_tiling_on_sc=True)

    @functools.partial(
        pl.kernel,
        mesh=_mesh,
        out_type=[jax.ShapeDtypeStruct((D * SU,), jnp.float32),
                  jax.ShapeDtypeStruct((D * SM,), jnp.float32)],
        scratch_types=[
            pltpu.VMEM((8, SLAB), jnp.float32),
            pltpu.VMEM((D * CHUNK,), jnp.float32),
            pltpu.SemaphoreType.DMA,
            pltpu.SemaphoreType.DMA,
        ],
        compiler_params=cp,
    )
    def k(utt_hbm, mtt_hbm, utail_hbm, mtail_hbm, uo_hbm, mo_hbm,
          buf, tbuf, rsem, wsem):
        wid = lax.axis_index("s") * NC + lax.axis_index("c")

        def move_slab(src_hbm, dst_hbm, stride, g, c0, width):
            pltpu.async_copy(
                src_hbm.at[pl.ds(8 * g, 8), pl.ds(c0, width)],
                buf.at[:, pl.ds(0, width)], rsem).wait()
            for l in range(8):
                @pl.loop(0, width // CHUNK)
                def _(c):
                    off = pl.multiple_of(c * CHUNK, CHUNK)
                    pltpu.async_copy(
                        buf.at[l, pl.ds(off, CHUNK)],
                        dst_hbm.at[pl.ds(
                            (8 * g + l) * stride + c0 + off, CHUNK)],
                        wsem)
            pltpu.make_async_copy(
                src_hbm.at[pl.ds(8 * g, 8), pl.ds(c0, width)],
                buf.at[:, pl.ds(0, width)], wsem).wait()

        for g in range(4):
            @pl.loop(0, (USLAB + NW - 1) // NW)
            def _(kk):
                s = wid + NW * kk

                @pl.when(s < USLAB)
                def _():
                    c0 = pl.multiple_of(s * SLAB, SLAB)
                    move_slab(utt_hbm, uo_hbm, SU, g, c0, SLAB)

        for g in range(4):
            @pl.loop(0, (MSLAB + NW - 1) // NW)
            def _(kk):
                s = wid + NW * kk

                @pl.when(s < MSLAB)
                def _():
                    c0 = pl.multiple_of(s * SLAB, SLAB)
                    move_slab(mtt_hbm, mo_hbm, SM, g, c0, SLAB)

        for g in range(4):
            @pl.when(wid == 16 + g)
            def _():
                move_slab(utt_hbm, uo_hbm, SU, g, USLAB * SLAB, UREM)

            @pl.when(wid == 20 + g)
            def _():
                move_slab(mtt_hbm, mo_hbm, SM, g, MSLAB * SLAB, MREM)

        @pl.when(wid == 24)
        def _():
            pltpu.async_copy(utail_hbm, tbuf, rsem).wait()
            for j in range(D):
                pltpu.async_copy(
                    tbuf.at[pl.ds(j * CHUNK, CHUNK)],
                    uo_hbm.at[pl.ds(j * SU + FU, CHUNK)], wsem)
            pltpu.make_async_copy(utail_hbm, tbuf, wsem).wait()

        @pl.when(wid == 25)
        def _():
            pltpu.async_copy(mtail_hbm, tbuf, rsem).wait()
            for j in range(D):
                pltpu.async_copy(
                    tbuf.at[pl.ds(j * CHUNK, CHUNK)],
                    mo_hbm.at[pl.ds(j * SM + FM, CHUNK)], wsem)
            pltpu.make_async_copy(mtail_hbm, tbuf, wsem).wait()

    return k


def _make_gather_kernel():
    cp = pltpu.CompilerParams(needs_layout_passes=False,
                              use_tc_tiling_on_sc=False)

    @functools.partial(
        pl.kernel,
        mesh=_mesh,
        out_type=jax.ShapeDtypeStruct((B,), jnp.float32),
        scratch_types=[
            pltpu.VMEM((NCHUNK, CHUNK), jnp.int32),
            pltpu.VMEM((NCHUNK, CHUNK), jnp.int32),
            pltpu.VMEM((NCHUNK, CHUNK), jnp.int32),
            pltpu.VMEM((NCHUNK, CHUNK), jnp.int32),
            pltpu.VMEM((D, BPW), jnp.float32),
            pltpu.VMEM((D, BPW), jnp.float32),
            pltpu.VMEM((F, BPW), jnp.float32),
            pltpu.VMEM((80,), jnp.float32),
            pltpu.VMEM((BPW,), jnp.float32),
            pltpu.SemaphoreType.DMA,
        ],
        compiler_params=cp,
    )
    def k(ui_hbm, mi_hbm, mft_hbm, ut_hbm, mt_hbm, wb_hbm, o_hbm,
          ui_v, mi_v, uf_v, mg_v, ue_v, me_v, mf_v, wb_v, o_v, sem):
        wid = lax.axis_index("s") * NC + lax.axis_index("c")
        base = wid * BPW

        pltpu.sync_copy(ui_hbm.at[pl.ds(wid * NCHUNK, NCHUNK)], ui_v)
        pltpu.sync_copy(mi_hbm.at[pl.ds(wid * NCHUNK, NCHUNK)], mi_v)

        @pl.loop(0, D)
        def _(j):
            for c in range(NCHUNK):
                for s in range(CHUNK // L):
                    sl = pl.ds(s * L, L)
                    uf_v[c, sl] = ui_v[c, sl] + j * SU
                    mg_v[c, sl] = mi_v[c, sl] + j * SM
            for c in range(NCHUNK):
                dst = pl.ds(c * CHUNK, CHUNK)
                pltpu.async_copy(
                    ut_hbm.at[uf_v.at[c]], ue_v.at[j].at[dst], sem)
                pltpu.async_copy(
                    mt_hbm.at[mg_v.at[c]], me_v.at[j].at[dst], sem)
            for c in range(NCHUNK):
                dst = pl.ds(c * CHUNK, CHUNK)
                pltpu.make_async_copy(
                    ut_hbm.at[uf_v.at[c]], ue_v.at[j].at[dst], sem).wait()
                pltpu.make_async_copy(
                    mt_hbm.at[mg_v.at[c]], me_v.at[j].at[dst], sem).wait()

        pltpu.sync_copy(mft_hbm.at[:, pl.ds(base, BPW)], mf_v)
        pltpu.sync_copy(wb_hbm, wb_v)

        wvecs = [wb_v[pl.ds(g * L, L)] for g in range(5)]
        bias = wvecs[4][0]

        @pl.loop(0, BPW // L)
        def _(g):
            sl = pl.ds(g * L, L)
            acc = jnp.full((L,), bias, jnp.float32)
            for j in range(D):
                w1j = wvecs[j // L][j % L]
                acc = acc + ue_v[j, sl] * me_v[j, sl] * w1j
            for f in range(F):
                w2f = wvecs[2 + f // L][f % L]
                acc = acc + mf_v[f, sl] * w2f
            o_v[sl] = acc

        pltpu.sync_copy(o_v, o_hbm.at[pl.ds(base, BPW)])

    return k


_sc_detile = _make_detile_kernel()
_sc_forward = _make_gather_kernel()


def kernel(user_idx, movie_idx, movie_feats, user_table, movie_table, W, b):
    ui = user_idx.astype(jnp.int32).reshape(B // CHUNK, CHUNK)
    mi = movie_idx.astype(jnp.int32).reshape(B // CHUNK, CHUNK)
    wb = jnp.zeros((80,), jnp.float32).at[:64].set(W[:, 0]).at[64].set(b[0])
    utail = jnp.pad(user_table[FU:].T, ((0, 0), (0, CHUNK - RU))).reshape(-1)
    mtail = jnp.pad(movie_table[FM:].T, ((0, 0), (0, CHUNK - RM))).reshape(-1)
    uflat, mflat = _sc_detile(user_table.T, movie_table.T, utail, mtail)
    return _sc_forward(ui, mi, movie_feats.T, uflat, mflat, wb)
